# baseline (speedup 1.0000x reference)
import jax
import jax.numpy as jnp
from jax import lax
from jax.experimental import pallas as pl
from jax.experimental.pallas import tpu as pltpu
from jax.experimental.pallas import tpu_sc as plsc

N = 10000
E = 320000
D = 128

NC = 2
NS = 16
NW = NC * NS
EPW = E // NW
C = 40
NCHUNK = EPW // C
UNROLL = 4
NLD = 10
RPS = N // NLD
MASK_HI = jnp.int32(-65536)


def _sc_kernel_body(r16_hbm, e_hbm, src_hbm, dst_hbm, z_hbm, out_hbm,
                    acc_sh, idx, e_v, g_v, m_v, *sems):
    c = lax.axis_index("c")
    s = lax.axis_index("s")
    wid = c * NS + s
    sem_ix = sems[0:4]
    sem_ld = sems[4:6]
    sem_sc = sems[6:8]

    def idx_descs(k, sl):
        base = wid * EPW + k * C
        return (
            pltpu.make_async_copy(dst_hbm.at[pl.ds(base, C)],
                                  idx.at[sl, pl.ds(0, C)], sem_ix[sl]),
            pltpu.make_async_copy(src_hbm.at[pl.ds(base, C)],
                                  idx.at[sl, pl.ds(C, C)], sem_ix[sl]),
        )

    def load_descs(k, b, sl):
        base = wid * EPW + k * C
        return (
            pltpu.make_async_copy(e_hbm.at[pl.ds(base, C)], e_v.at[b],
                                  sem_ld[b]),
            pltpu.make_async_copy(r16_hbm.at[idx.at[sl]], g_v.at[b],
                                  sem_ld[b]),
        )

    def scat_desc(b, sl):
        return pltpu.make_async_copy(m_v.at[b], acc_sh.at[idx.at[sl]],
                                     sem_sc[b])

    for d in idx_descs(0, 0):
        d.start()
    for d in idx_descs(1, 1):
        d.start()
    for d in idx_descs(0, 0):
        d.wait()
    for d in load_descs(0, 0, 0):
        d.start()

    @pl.when(s < NLD)
    def _zero():
        pltpu.sync_copy(z_hbm.at[pl.ds(s * RPS, RPS)],
                        acc_sh.at[pl.ds(s * RPS, RPS)])

    plsc.subcore_barrier()

    def do_chunk(k, j, in_loop):
        b2 = j % 2
        b4 = j % 4

        def _drain():
            scat_desc(b2, (b4 + 2) % 4).wait()

        if in_loop:
            pl.when(k >= 2)(_drain)
        else:
            _drain()

        if in_loop or k + 2 < NCHUNK:
            for d in idx_descs(k + 2, (b4 + 2) % 4):
                d.start()

        if in_loop or k + 1 < NCHUNK:
            for d in idx_descs(k + 1, (b4 + 1) % 4):
                d.wait()
            for d in load_descs(k + 1, (b2 + 1) % 2, (b4 + 1) % 4):
                d.start()

        for d in load_descs(k, b2, b4):
            d.wait()

        ev_r = e_v.at[b2]
        g_r = g_v.at[b2]
        m_r = m_v.at[b2]

        @plsc.parallel_loop(0, C, 1, unroll=4)
        def mul_body(i):
            for jj in range(D // 32):
                lo = pl.ds(jj * 32, 16)
                hi = pl.ds(jj * 32 + 16, 16)
                e_lo = ev_r[i, lo]
                e_hi = ev_r[i, hi]
                s_w = g_r[C + i, pl.ds(jj * 16, 16)]
                d_w = g_r[i, pl.ds(jj * 16, 16)]
                s_lo = lax.bitcast_convert_type(s_w << 16, jnp.float32)
                s_hi = lax.bitcast_convert_type(s_w & MASK_HI, jnp.float32)
                d_lo = lax.bitcast_convert_type(d_w << 16, jnp.float32)
                d_hi = lax.bitcast_convert_type(d_w & MASK_HI, jnp.float32)
                m_r[i, lo] = s_lo * e_lo
                m_r[i, hi] = s_hi * e_hi
                m_r[C + i, lo] = d_lo * e_lo
                m_r[C + i, hi] = d_hi * e_hi

        pltpu.async_copy(m_v.at[b2], acc_sh.at[idx.at[b4]], sem_sc[b2],
                         add=True)

    NTAIL = NCHUNK % UNROLL
    NMAIN = NCHUNK - NTAIL

    @pl.loop(0, NMAIN, step=UNROLL)
    def _trips(k0):
        for j in range(UNROLL):
            do_chunk(k0 + j, j, True)

    for k in range(NMAIN, NCHUNK):
        do_chunk(k, k % UNROLL, False)

    k1, k2 = NCHUNK - 2, NCHUNK - 1
    scat_desc(k1 % 2, k1 % 4).wait()
    scat_desc(k2 % 2, k2 % 4).wait()

    plsc.subcore_barrier()

    @pl.when(s < NLD)
    def _dump():
        pltpu.sync_copy(acc_sh.at[pl.ds(s * RPS, RPS)],
                        out_hbm.at[c, pl.ds(s * RPS, RPS)])


@jax.jit
def _message_passing_sc(r16, e, src, dst, z):
    mesh = plsc.VectorSubcoreMesh(core_axis_name="c", subcore_axis_name="s")
    partials = pl.kernel(
        _sc_kernel_body,
        out_type=jax.ShapeDtypeStruct((NC, N, D), jnp.float32),
        mesh=mesh,
        compiler_params=pltpu.CompilerParams(use_tc_tiling_on_sc=False),
        scratch_types=[
            pltpu.VMEM_SHARED((N, D), jnp.float32),
            pltpu.VMEM((4, 2 * C), jnp.int32),
            pltpu.VMEM((2, C, D), jnp.float32),
            pltpu.VMEM((2, 2 * C, D // 2), jnp.int32),
            pltpu.VMEM((2, 2 * C, D), jnp.float32),
        ] + [pltpu.SemaphoreType.DMA] * 8,
    )(r16, e, src, dst, z)
    return partials


def _add_body(a_ref, b_ref, o_ref):
    o_ref[...] = a_ref[...] + b_ref[...]


def _combine_tc(partials):
    return pl.pallas_call(
        _add_body,
        out_shape=jax.ShapeDtypeStruct((N, D), jnp.float32),
        grid=(10,),
        in_specs=[
            pl.BlockSpec((N // 10, D), lambda i: (i, 0)),
            pl.BlockSpec((N // 10, D), lambda i: (i, 0)),
        ],
        out_specs=pl.BlockSpec((N // 10, D), lambda i: (i, 0)),
    )(partials[0], partials[1])


def kernel(r, e, a):
    a = a.astype(jnp.int32)
    src = a[:, 0]
    dst = a[:, 1]
    r16 = (r.reshape(N, D // 32, 2, 16)
             .transpose(0, 1, 3, 2)
             .reshape(N, D // 2, 2)
             .astype(jnp.bfloat16))
    r16 = lax.bitcast_convert_type(r16, jnp.int32)
    z = jnp.zeros((N, D), jnp.float32)
    partials = _message_passing_sc(r16, e, src, dst, z)
    return _combine_tc(partials)


# speedup vs baseline: 1.0218x; 1.0218x over previous
import jax
import jax.numpy as jnp
from jax import lax
from jax.experimental import pallas as pl
from jax.experimental.pallas import tpu as pltpu
from jax.experimental.pallas import tpu_sc as plsc

N = 10000
E = 320000
D = 128

NC = 2
NS = 16
NW = NC * NS
EPW = E // NW
C = 40
NCHUNK = EPW // C
UNROLL = 4
NLD = 10
RPS = N // NLD
MASK_HI = jnp.int32(-65536)


def _sc_kernel_body(r16_hbm, e_hbm, src_hbm, dst_hbm, out_hbm,
                    acc_sh, idx, e_v, g_v, m_v, *sems):
    c = lax.axis_index("c")
    s = lax.axis_index("s")
    wid = c * NS + s
    sem_ix = sems[0:4]
    sem_ld = sems[4:6]
    sem_sc = sems[6:8]

    def idx_descs(k, sl):
        base = wid * EPW + k * C
        return (
            pltpu.make_async_copy(dst_hbm.at[pl.ds(base, C)],
                                  idx.at[sl, pl.ds(0, C)], sem_ix[sl]),
            pltpu.make_async_copy(src_hbm.at[pl.ds(base, C)],
                                  idx.at[sl, pl.ds(C, C)], sem_ix[sl]),
        )

    def load_descs(k, b, sl):
        base = wid * EPW + k * C
        return (
            pltpu.make_async_copy(e_hbm.at[pl.ds(base, C)], e_v.at[b],
                                  sem_ld[b]),
            pltpu.make_async_copy(r16_hbm.at[idx.at[sl]], g_v.at[b],
                                  sem_ld[b]),
        )

    def scat_desc(b, sl):
        return pltpu.make_async_copy(m_v.at[b], acc_sh.at[idx.at[sl]],
                                     sem_sc[b])

    for d in idx_descs(0, 0):
        d.start()
    for d in idx_descs(1, 1):
        d.start()
    for d in idx_descs(0, 0):
        d.wait()
    for d in load_descs(0, 0, 0):
        d.start()

    zrow = jnp.zeros((16,), jnp.float32)

    @plsc.parallel_loop(0, 2 * C, 1, unroll=2)
    def _zfill(i):
        for jj in range(D // 16):
            m_v[0, i, pl.ds(jj * 16, 16)] = zrow

    zbase = s * (N // NS)
    for blk in range(7):
        pltpu.async_copy(m_v.at[0], acc_sh.at[pl.ds(zbase + blk * 80, 80)],
                         sem_sc[0])
    pltpu.async_copy(m_v.at[0, pl.ds(0, 65)],
                     acc_sh.at[pl.ds(zbase + 560, 65)], sem_sc[0])
    for blk in range(7):
        pltpu.make_async_copy(m_v.at[0], acc_sh.at[pl.ds(zbase + blk * 80, 80)],
                              sem_sc[0]).wait()
    pltpu.make_async_copy(m_v.at[0, pl.ds(0, 65)],
                          acc_sh.at[pl.ds(zbase + 560, 65)], sem_sc[0]).wait()

    plsc.subcore_barrier()

    def do_chunk(k, j, in_loop):
        b2 = j % 2
        b4 = j % 4

        def _drain():
            scat_desc(b2, (b4 + 2) % 4).wait()

        if in_loop:
            pl.when(k >= 2)(_drain)
        else:
            _drain()

        if in_loop or k + 2 < NCHUNK:
            for d in idx_descs(k + 2, (b4 + 2) % 4):
                d.start()

        if in_loop or k + 1 < NCHUNK:
            for d in idx_descs(k + 1, (b4 + 1) % 4):
                d.wait()
            for d in load_descs(k + 1, (b2 + 1) % 2, (b4 + 1) % 4):
                d.start()

        for d in load_descs(k, b2, b4):
            d.wait()

        ev_r = e_v.at[b2]
        g_r = g_v.at[b2]
        m_r = m_v.at[b2]

        @plsc.parallel_loop(0, C, 1, unroll=4)
        def mul_body(i):
            for jj in range(D // 32):
                lo = pl.ds(jj * 32, 16)
                hi = pl.ds(jj * 32 + 16, 16)
                e_lo = ev_r[i, lo]
                e_hi = ev_r[i, hi]
                s_w = g_r[C + i, pl.ds(jj * 16, 16)]
                d_w = g_r[i, pl.ds(jj * 16, 16)]
                s_lo = lax.bitcast_convert_type(s_w << 16, jnp.float32)
                s_hi = lax.bitcast_convert_type(s_w & MASK_HI, jnp.float32)
                d_lo = lax.bitcast_convert_type(d_w << 16, jnp.float32)
                d_hi = lax.bitcast_convert_type(d_w & MASK_HI, jnp.float32)
                m_r[i, lo] = s_lo * e_lo
                m_r[i, hi] = s_hi * e_hi
                m_r[C + i, lo] = d_lo * e_lo
                m_r[C + i, hi] = d_hi * e_hi

        pltpu.async_copy(m_v.at[b2], acc_sh.at[idx.at[b4]], sem_sc[b2],
                         add=True)

    NTAIL = NCHUNK % UNROLL
    NMAIN = NCHUNK - NTAIL

    @pl.loop(0, NMAIN, step=UNROLL)
    def _trips(k0):
        for j in range(UNROLL):
            do_chunk(k0 + j, j, True)

    for k in range(NMAIN, NCHUNK):
        do_chunk(k, k % UNROLL, False)

    k1, k2 = NCHUNK - 2, NCHUNK - 1
    scat_desc(k1 % 2, k1 % 4).wait()
    scat_desc(k2 % 2, k2 % 4).wait()

    plsc.subcore_barrier()

    @pl.when(s < NLD)
    def _dump():
        pltpu.sync_copy(acc_sh.at[pl.ds(s * RPS, RPS)],
                        out_hbm.at[c, pl.ds(s * RPS, RPS)])


@jax.jit
def _message_passing_sc(r16, e, src, dst):
    mesh = plsc.VectorSubcoreMesh(core_axis_name="c", subcore_axis_name="s")
    partials = pl.kernel(
        _sc_kernel_body,
        out_type=jax.ShapeDtypeStruct((NC, N, D), jnp.float32),
        mesh=mesh,
        compiler_params=pltpu.CompilerParams(use_tc_tiling_on_sc=False),
        scratch_types=[
            pltpu.VMEM_SHARED((N, D), jnp.float32),
            pltpu.VMEM((4, 2 * C), jnp.int32),
            pltpu.VMEM((2, C, D), jnp.float32),
            pltpu.VMEM((2, 2 * C, D // 2), jnp.int32),
            pltpu.VMEM((2, 2 * C, D), jnp.float32),
        ] + [pltpu.SemaphoreType.DMA] * 8,
    )(r16, e, src, dst)
    return partials


def _add_body(a_ref, b_ref, o_ref):
    o_ref[...] = a_ref[...] + b_ref[...]


def _combine_tc(partials):
    return pl.pallas_call(
        _add_body,
        out_shape=jax.ShapeDtypeStruct((N, D), jnp.float32),
        grid=(10,),
        in_specs=[
            pl.BlockSpec((N // 10, D), lambda i: (i, 0)),
            pl.BlockSpec((N // 10, D), lambda i: (i, 0)),
        ],
        out_specs=pl.BlockSpec((N // 10, D), lambda i: (i, 0)),
    )(partials[0], partials[1])


def kernel(r, e, a):
    a = a.astype(jnp.int32)
    src = a[:, 0]
    dst = a[:, 1]
    r16 = (r.reshape(N, D // 32, 2, 16)
             .transpose(0, 1, 3, 2)
             .reshape(N, D // 2, 2)
             .astype(jnp.bfloat16))
    r16 = lax.bitcast_convert_type(r16, jnp.int32)
    partials = _message_passing_sc(r16, e, src, dst)
    return _combine_tc(partials)


# speedup vs baseline: 1.0229x; 1.0011x over previous
import jax
import jax.numpy as jnp
from jax import lax
from jax.experimental import pallas as pl
from jax.experimental.pallas import tpu as pltpu
from jax.experimental.pallas import tpu_sc as plsc

N = 10000
E = 320000
D = 128

NC = 2
NS = 16
NW = NC * NS
EPW = E // NW
C = 40
NCHUNK = EPW // C
UNROLL = 4
NLD = 10
RPS = N // NLD
MASK_HI = jnp.int32(-65536)


def _sc_kernel_body(r16_hbm, e_hbm, src_hbm, dst_hbm, out_hbm,
                    acc_sh, idx, e_v, g_v, m_v, *sems):
    c = lax.axis_index("c")
    s = lax.axis_index("s")
    wid = c * NS + s
    sem_ix = sems[0:4]
    sem_ld = sems[4:6]
    sem_sc = sems[6:8]

    def idx_descs(k, sl):
        base = wid * EPW + k * C
        return (
            pltpu.make_async_copy(dst_hbm.at[pl.ds(base, C)],
                                  idx.at[sl, pl.ds(0, C)], sem_ix[sl]),
            pltpu.make_async_copy(src_hbm.at[pl.ds(base, C)],
                                  idx.at[sl, pl.ds(C, C)], sem_ix[sl]),
        )

    def load_descs(k, b, sl):
        base = wid * EPW + k * C
        return (
            pltpu.make_async_copy(e_hbm.at[pl.ds(base, C)], e_v.at[b],
                                  sem_ld[b]),
            pltpu.make_async_copy(r16_hbm.at[idx.at[sl]], g_v.at[b],
                                  sem_ld[b]),
        )

    def scat_desc(b, sl):
        return pltpu.make_async_copy(m_v.at[b], acc_sh.at[idx.at[sl]],
                                     sem_sc[b])

    for d in idx_descs(0, 0):
        d.start()
    for d in idx_descs(1, 1):
        d.start()
    for d in idx_descs(0, 0):
        d.wait()
    for d in load_descs(0, 0, 0):
        d.start()

    zrow = jnp.zeros((16,), jnp.float32)

    @plsc.parallel_loop(0, 2 * C, 1, unroll=2)
    def _zfill(i):
        for jj in range(D // 16):
            m_v[0, i, pl.ds(jj * 16, 16)] = zrow

    zbase = s * (N // NS)
    for blk in range(7):
        pltpu.async_copy(m_v.at[0], acc_sh.at[pl.ds(zbase + blk * 80, 80)],
                         sem_sc[0])
    pltpu.async_copy(m_v.at[0, pl.ds(0, 65)],
                     acc_sh.at[pl.ds(zbase + 560, 65)], sem_sc[0])
    for blk in range(7):
        pltpu.make_async_copy(m_v.at[0], acc_sh.at[pl.ds(zbase + blk * 80, 80)],
                              sem_sc[0]).wait()
    pltpu.make_async_copy(m_v.at[0, pl.ds(0, 65)],
                          acc_sh.at[pl.ds(zbase + 560, 65)], sem_sc[0]).wait()

    plsc.subcore_barrier()

    def do_chunk(k, j, in_loop):
        b2 = j % 2
        b4 = j % 4

        def _drain():
            scat_desc(b2, (b4 + 2) % 4).wait()

        if in_loop:
            pl.when(k >= 2)(_drain)
        else:
            _drain()

        if in_loop or k + 2 < NCHUNK:
            for d in idx_descs(k + 2, (b4 + 2) % 4):
                d.start()

        if in_loop or k + 1 < NCHUNK:
            for d in idx_descs(k + 1, (b4 + 1) % 4):
                d.wait()
            for d in load_descs(k + 1, (b2 + 1) % 2, (b4 + 1) % 4):
                d.start()

        for d in load_descs(k, b2, b4):
            d.wait()

        ev_r = e_v.at[b2]
        g_r = g_v.at[b2]
        m_r = m_v.at[b2]

        @plsc.parallel_loop(0, C, 1, unroll=4)
        def mul_body(i):
            for jj in range(D // 32):
                lo = pl.ds(jj * 32, 16)
                hi = pl.ds(jj * 32 + 16, 16)
                e_lo = ev_r[i, lo]
                e_hi = ev_r[i, hi]
                s_w = g_r[C + i, pl.ds(jj * 16, 16)]
                d_w = g_r[i, pl.ds(jj * 16, 16)]
                s_lo = lax.bitcast_convert_type(s_w << 16, jnp.float32)
                s_hi = lax.bitcast_convert_type(s_w & MASK_HI, jnp.float32)
                d_lo = lax.bitcast_convert_type(d_w << 16, jnp.float32)
                d_hi = lax.bitcast_convert_type(d_w & MASK_HI, jnp.float32)
                m_r[i, lo] = s_lo * e_lo
                m_r[i, hi] = s_hi * e_hi
                m_r[C + i, lo] = d_lo * e_lo
                m_r[C + i, hi] = d_hi * e_hi

        pltpu.async_copy(m_v.at[b2], acc_sh.at[idx.at[b4]], sem_sc[b2],
                         add=True)

    NTAIL = NCHUNK % UNROLL
    NMAIN = NCHUNK - NTAIL

    @pl.loop(0, NMAIN, step=UNROLL)
    def _trips(k0):
        for j in range(UNROLL):
            do_chunk(k0 + j, j, True)

    for k in range(NMAIN, NCHUNK):
        do_chunk(k, k % UNROLL, False)

    k1, k2 = NCHUNK - 2, NCHUNK - 1
    scat_desc(k1 % 2, k1 % 4).wait()
    scat_desc(k2 % 2, k2 % 4).wait()

    plsc.subcore_barrier()

    pltpu.sync_copy(acc_sh.at[pl.ds(zbase, N // NS)],
                    out_hbm.at[c, pl.ds(zbase, N // NS)])


@jax.jit
def _message_passing_sc(r16, e, src, dst):
    mesh = plsc.VectorSubcoreMesh(core_axis_name="c", subcore_axis_name="s")
    partials = pl.kernel(
        _sc_kernel_body,
        out_type=jax.ShapeDtypeStruct((NC, N, D), jnp.float32),
        mesh=mesh,
        compiler_params=pltpu.CompilerParams(use_tc_tiling_on_sc=False),
        scratch_types=[
            pltpu.VMEM_SHARED((N, D), jnp.float32),
            pltpu.VMEM((4, 2 * C), jnp.int32),
            pltpu.VMEM((2, C, D), jnp.float32),
            pltpu.VMEM((2, 2 * C, D // 2), jnp.int32),
            pltpu.VMEM((2, 2 * C, D), jnp.float32),
        ] + [pltpu.SemaphoreType.DMA] * 8,
    )(r16, e, src, dst)
    return partials


def _add_body(a_ref, b_ref, o_ref):
    o_ref[...] = a_ref[...] + b_ref[...]


def _combine_tc(partials):
    return pl.pallas_call(
        _add_body,
        out_shape=jax.ShapeDtypeStruct((N, D), jnp.float32),
        grid=(10,),
        in_specs=[
            pl.BlockSpec((N // 10, D), lambda i: (i, 0)),
            pl.BlockSpec((N // 10, D), lambda i: (i, 0)),
        ],
        out_specs=pl.BlockSpec((N // 10, D), lambda i: (i, 0)),
    )(partials[0], partials[1])


def kernel(r, e, a):
    a = a.astype(jnp.int32)
    src = a[:, 0]
    dst = a[:, 1]
    r16 = (r.reshape(N, D // 32, 2, 16)
             .transpose(0, 1, 3, 2)
             .reshape(N, D // 2, 2)
             .astype(jnp.bfloat16))
    r16 = lax.bitcast_convert_type(r16, jnp.int32)
    partials = _message_passing_sc(r16, e, src, dst)
    return _combine_tc(partials)
